# Initial kernel scaffold; baseline (speedup 1.0000x reference)
import jax
import jax.numpy as jnp
from jax.experimental import pallas as pl


def kernel(spins, edge_idx_i, edge_idx_j, linear, quadratic):
    raise NotImplementedError("write your pallas kernel here")


# speedup vs baseline: 3.6055x; 3.6055x over previous
import functools

import jax
import jax.numpy as jnp
from jax import lax
from jax.experimental import pallas as pl
from jax.experimental.pallas import tpu as pltpu
from jax.experimental.pallas import tpu_sc as plsc

N_NODES = 10000
N_EDGES = 160000
BATCH = 256

L = 16
NC = 2
NS = 16
NW = NC * NS
ROWS = BATCH // NW
CHUNK = 8000
N_CHUNKS = N_EDGES // CHUNK


def _energy_body(spins_hbm, ii_hbm, jj_hbm, lin_hbm, q_hbm, out_hbm,
                 s_v, lin_v, iv_v, jv_v, qv_v, ob_v):
    wid = lax.axis_index("s") * NC + lax.axis_index("c")
    base = wid * (ROWS * N_NODES)
    pltpu.sync_copy(spins_hbm.at[pl.ds(base, ROWS * N_NODES)], s_v)
    pltpu.sync_copy(lin_hbm, lin_v)

    def lin_step(v, accs):
        lv = lin_v[pl.ds(v * L, L)]
        return tuple(accs[r] + s_v[pl.ds(r * N_NODES + v * L, L)] * lv
                     for r in range(ROWS))

    accs = tuple(jnp.zeros((L,), jnp.float32) for _ in range(ROWS))
    accs = lax.fori_loop(0, N_NODES // L, lin_step, accs)

    for c in range(N_CHUNKS):
        pltpu.sync_copy(ii_hbm.at[pl.ds(c * CHUNK, CHUNK)], iv_v)
        pltpu.sync_copy(jj_hbm.at[pl.ds(c * CHUNK, CHUNK)], jv_v)
        pltpu.sync_copy(q_hbm.at[pl.ds(c * CHUNK, CHUNK)], qv_v)

        def edge_step(v, accs):
            iv = iv_v[pl.ds(v * L, L)]
            jv = jv_v[pl.ds(v * L, L)]
            qv = qv_v[pl.ds(v * L, L)]
            new = []
            for r in range(ROWS):
                a = plsc.load_gather(s_v, [iv + r * N_NODES])
                b = plsc.load_gather(s_v, [jv + r * N_NODES])
                new.append(accs[r] + qv * (a * b))
            return tuple(new)

        accs = lax.fori_loop(0, CHUNK // L, edge_step, accs)

    lane = lax.iota(jnp.int32, L)
    ob = jnp.zeros((L,), jnp.float32)
    for r in range(ROWS):
        ob = jnp.where(lane == r, jnp.sum(accs[r]), ob)
    ob_v[...] = ob
    pltpu.sync_copy(ob_v, out_hbm.at[wid])


_energy_kernel = functools.partial(
    pl.kernel,
    out_type=jax.ShapeDtypeStruct((NW, L), jnp.float32),
    mesh=plsc.VectorSubcoreMesh(core_axis_name="c", subcore_axis_name="s"),
    compiler_params=pltpu.CompilerParams(needs_layout_passes=False),
    scratch_types=[
        pltpu.VMEM((ROWS * N_NODES,), jnp.float32),
        pltpu.VMEM((N_NODES,), jnp.float32),
        pltpu.VMEM((CHUNK,), jnp.int32),
        pltpu.VMEM((CHUNK,), jnp.int32),
        pltpu.VMEM((CHUNK,), jnp.float32),
        pltpu.VMEM((L,), jnp.float32),
    ],
)(_energy_body)


def kernel(spins, edge_idx_i, edge_idx_j, linear, quadratic):
    out2d = _energy_kernel(spins.reshape(-1), edge_idx_i.astype(jnp.int32),
                           edge_idx_j.astype(jnp.int32), linear, quadratic)
    return out2d[:, :ROWS].reshape(BATCH)


# speedup vs baseline: 4.1726x; 1.1573x over previous
import functools

import jax
import jax.numpy as jnp
from jax import lax
from jax.experimental import pallas as pl
from jax.experimental.pallas import tpu as pltpu
from jax.experimental.pallas import tpu_sc as plsc

N_NODES = 10000
N_EDGES = 160000
BATCH = 256

L = 16
NC = 2
NS = 16
NW = NC * NS
ROWS = BATCH // NW
CHUNK = 6400
N_CHUNKS = N_EDGES // CHUNK


def _energy_body(spins_hbm, ii_hbm, jj_hbm, lin_hbm, q_hbm, out_hbm,
                 s_v, lin_v, iv_v, jv_v, qv_v, ob_v, sem0, sem1):
    wid = lax.axis_index("s") * NC + lax.axis_index("c")
    base = wid * (ROWS * N_NODES)

    sems = (sem0, sem1)

    def fire(c):
        slot = c % 2
        off = c * CHUNK
        return (
            pltpu.async_copy(ii_hbm.at[pl.ds(off, CHUNK)], iv_v.at[slot], sems[slot]),
            pltpu.async_copy(jj_hbm.at[pl.ds(off, CHUNK)], jv_v.at[slot], sems[slot]),
            pltpu.async_copy(q_hbm.at[pl.ds(off, CHUNK)], qv_v.at[slot], sems[slot]),
        )

    inflight = {0: fire(0), 1: fire(1)}

    pltpu.sync_copy(spins_hbm.at[pl.ds(base, ROWS * N_NODES)], s_v)
    pltpu.sync_copy(lin_hbm, lin_v)

    def lin_step(v, accs):
        lv = lin_v[pl.ds(v * L, L)]
        return tuple(accs[r] + s_v[pl.ds(r * N_NODES + v * L, L)] * lv
                     for r in range(ROWS))

    accs = tuple(jnp.zeros((L,), jnp.float32) for _ in range(ROWS))
    accs = lax.fori_loop(0, N_NODES // L, lin_step, accs)

    for c in range(N_CHUNKS):
        slot = c % 2
        for d in inflight.pop(c):
            d.wait()

        def edge_step(v, accs):
            iv = iv_v[slot, pl.ds(v * L, L)]
            jv = jv_v[slot, pl.ds(v * L, L)]
            qv = qv_v[slot, pl.ds(v * L, L)]
            new = []
            for r in range(ROWS):
                a = plsc.load_gather(s_v, [iv + r * N_NODES])
                b = plsc.load_gather(s_v, [jv + r * N_NODES])
                new.append(accs[r] + qv * (a * b))
            return tuple(new)

        accs = lax.fori_loop(0, CHUNK // L, edge_step, accs)

        if c + 2 < N_CHUNKS:
            inflight[c + 2] = fire(c + 2)

    lane = lax.iota(jnp.int32, L)
    ob = jnp.zeros((L,), jnp.float32)
    for r in range(ROWS):
        ob = jnp.where(lane == r, jnp.sum(accs[r]), ob)
    ob_v[...] = ob
    pltpu.sync_copy(ob_v, out_hbm.at[wid])


_energy_kernel = functools.partial(
    pl.kernel,
    out_type=jax.ShapeDtypeStruct((NW, L), jnp.float32),
    mesh=plsc.VectorSubcoreMesh(core_axis_name="c", subcore_axis_name="s"),
    compiler_params=pltpu.CompilerParams(needs_layout_passes=False),
    scratch_types=[
        pltpu.VMEM((ROWS * N_NODES,), jnp.float32),
        pltpu.VMEM((N_NODES,), jnp.float32),
        pltpu.VMEM((2, CHUNK), jnp.int32),
        pltpu.VMEM((2, CHUNK), jnp.int32),
        pltpu.VMEM((2, CHUNK), jnp.float32),
        pltpu.VMEM((L,), jnp.float32),
        pltpu.SemaphoreType.DMA,
        pltpu.SemaphoreType.DMA,
    ],
)(_energy_body)


def kernel(spins, edge_idx_i, edge_idx_j, linear, quadratic):
    out2d = _energy_kernel(spins.reshape(-1), edge_idx_i.astype(jnp.int32),
                           edge_idx_j.astype(jnp.int32), linear, quadratic)
    return out2d[:, :ROWS].reshape(BATCH)
